# baseline (speedup 1.0000x reference)
import jax
import jax.numpy as jnp
from jax.experimental import pallas as pl

B, C_IN, N_T = 4, 384, 196
K, D = 1024, 64

_BF = jnp.bfloat16
_F32 = jnp.float32


def _split3(x):
    x0 = x.astype(_BF)
    r1 = x - x0.astype(_F32)
    x1 = r1.astype(_BF)
    x2 = (r1 - x1.astype(_F32)).astype(_BF)
    return x0, x1, x2


def _vq_body(z_ref, w_ref, emb_ref, out_ref):
    w = w_ref[...]
    emb = emb_ref[...]
    emb2 = jnp.sum(emb * emb, axis=1, keepdims=True)
    emb_norm = jnp.sqrt(emb2)

    e0, e1, e2 = _split3(emb)
    m2e0, m2e1, m2e2 = (-2.0 * e0.astype(_F32)).astype(_BF), \
                       (-2.0 * e1.astype(_F32)).astype(_BF), \
                       (-2.0 * e2.astype(_F32)).astype(_BF)
    a1 = jnp.concatenate([m2e0, m2e1, m2e2, m2e0], axis=1)
    q0, q1, q2 = _split3(emb2)
    onesk = jnp.ones((K, 3), _BF)
    a2 = jnp.concatenate([m2e0, m2e1, q0.astype(_BF), q1.astype(_BF),
                          q2.astype(_BF), onesk], axis=1)

    et0, et1, _ = _split3(emb.T)
    at = jnp.concatenate([et0, et1], axis=0)

    iota_k = jax.lax.broadcasted_iota(jnp.int32, (K, N_T), 0)
    KC = 128
    iota_c = jax.lax.broadcasted_iota(jnp.int32, (KC, N_T), 0)
    for b in range(B):
        zb = z_ref[b]
        ze = jnp.dot(w, zb)
        ze2 = jnp.sum(ze * ze, axis=0, keepdims=True)
        s0 = ze.astype(_BF)
        r1 = ze - s0.astype(_F32)
        s1 = r1.astype(_BF)
        r2 = r1 - s1.astype(_F32)
        t0, t1, t2 = _split3(ze2)
        one_n = jnp.ones((1, N_T), _F32)
        x1 = jnp.concatenate([ze, ze, ze, r2], axis=0).astype(_BF)
        x2 = jnp.concatenate([r1, r1, one_n, one_n, one_n,
                              t0.astype(_F32), t1.astype(_F32),
                              t2.astype(_F32)], axis=0).astype(_BF)
        num2 = (jnp.dot(a1, x1, preferred_element_type=_F32)
                + jnp.dot(a2, x2, preferred_element_type=_F32))
        a_n = jnp.sqrt(ze2)
        mval = None
        for c in range(K // KC):
            den_c = a_n + emb_norm[c * KC:(c + 1) * KC]
            s2c = num2[c * KC:(c + 1) * KC] / (den_c * den_c)
            cmin = jnp.min(s2c, axis=0, keepdims=True)
            cidx = jnp.min(jnp.where(s2c == cmin, iota_c, KC),
                           axis=0, keepdims=True) + (c * KC)
            if mval is None:
                mval, midx = cmin, cidx
            else:
                upd = cmin < mval
                midx = jnp.where(upd, cidx, midx)
                mval = jnp.minimum(mval, cmin)
        idx = midx
        onehot = (iota_k == idx).astype(_BF)
        zq2 = jnp.dot(at, onehot, preferred_element_type=_F32)
        zq = zq2[:D] + zq2[D:]
        out_ref[b] = ze + (zq - ze)


@jax.jit
def kernel(z, W, emb):
    return pl.pallas_call(
        _vq_body,
        out_shape=jax.ShapeDtypeStruct((B, D, N_T), jnp.float32),
    )(z, W, emb)


# speedup vs baseline: 1.0145x; 1.0145x over previous
import jax
import jax.numpy as jnp
from jax.experimental import pallas as pl
from jax.experimental.pallas import tpu as pltpu

B, C_IN, N_T = 4, 384, 196
K, D = 1024, 64

_BF = jnp.bfloat16
_F32 = jnp.float32


def _split3(x):
    x0 = x.astype(_BF)
    r1 = x - x0.astype(_F32)
    x1 = r1.astype(_BF)
    x2 = (r1 - x1.astype(_F32)).astype(_BF)
    return x0, x1, x2


def _vq_body(z_ref, w_ref, emb_ref, out_ref, a1_ref, a2_ref, at_ref, en_ref):
    @pl.when(pl.program_id(0) == 0)
    def _prep():
        emb = emb_ref[...]
        emb2 = jnp.sum(emb * emb, axis=1, keepdims=True)
        en_ref[...] = jnp.sqrt(emb2)
        e0, e1, e2 = _split3(emb)
        m2e0 = (-2.0 * e0.astype(_F32)).astype(_BF)
        m2e1 = (-2.0 * e1.astype(_F32)).astype(_BF)
        m2e2 = (-2.0 * e2.astype(_F32)).astype(_BF)
        a1_ref[...] = jnp.concatenate([m2e0, m2e1, m2e2, m2e0], axis=1)
        q0, q1, q2 = _split3(emb2)
        onesk = jnp.ones((K, 3), _BF)
        a2_ref[...] = jnp.concatenate([m2e0, m2e1, q0.astype(_BF),
                                       q1.astype(_BF), q2.astype(_BF),
                                       onesk], axis=1)
        et0, et1, _ = _split3(emb.T)
        at_ref[...] = jnp.concatenate([et0, et1], axis=0)

    a1 = a1_ref[...]
    a2 = a2_ref[...]
    emb_norm = en_ref[...]

    iota_k = jax.lax.broadcasted_iota(jnp.int32, (K, N_T), 0)
    KC = 256
    iota_c = jax.lax.broadcasted_iota(jnp.int32, (KC, N_T), 0)

    zb = z_ref[0]
    ze = jnp.dot(w_ref[...], zb)
    ze2 = jnp.sum(ze * ze, axis=0, keepdims=True)
    s0 = ze.astype(_BF)
    r1 = ze - s0.astype(_F32)
    s1 = r1.astype(_BF)
    r2 = r1 - s1.astype(_F32)
    t0, t1, t2 = _split3(ze2)
    one_n = jnp.ones((1, N_T), _F32)
    x1 = jnp.concatenate([ze, ze, ze, r2], axis=0).astype(_BF)
    x2 = jnp.concatenate([r1, r1, one_n, one_n, one_n,
                          t0.astype(_F32), t1.astype(_F32),
                          t2.astype(_F32)], axis=0).astype(_BF)
    num2 = (jnp.dot(a1, x1, preferred_element_type=_F32)
            + jnp.dot(a2, x2, preferred_element_type=_F32))
    a_n = jnp.sqrt(ze2)
    mval = None
    for c in range(K // KC):
        den_c = a_n + emb_norm[c * KC:(c + 1) * KC]
        s2c = num2[c * KC:(c + 1) * KC] / (den_c * den_c)
        cmin = jnp.min(s2c, axis=0, keepdims=True)
        cidx = jnp.min(jnp.where(s2c == cmin, iota_c, KC),
                       axis=0, keepdims=True) + (c * KC)
        if mval is None:
            mval, midx = cmin, cidx
        else:
            upd = cmin < mval
            midx = jnp.where(upd, cidx, midx)
            mval = jnp.minimum(mval, cmin)
    onehot = (iota_k == midx).astype(_BF)
    zq2 = jnp.dot(at_ref[...], onehot,
                  preferred_element_type=_F32)
    zq = zq2[:D] + zq2[D:]
    out_ref[0] = ze + (zq - ze)


@jax.jit
def kernel(z, W, emb):
    return pl.pallas_call(
        _vq_body,
        grid=(B,),
        in_specs=[
            pl.BlockSpec((1, C_IN, N_T), lambda b: (b, 0, 0)),
            pl.BlockSpec((D, C_IN), lambda b: (0, 0)),
            pl.BlockSpec((K, D), lambda b: (0, 0)),
        ],
        out_specs=pl.BlockSpec((1, D, N_T), lambda b: (b, 0, 0)),
        out_shape=jax.ShapeDtypeStruct((B, D, N_T), jnp.float32),
        scratch_shapes=[
            pltpu.VMEM((K, 4 * D), _BF),
            pltpu.VMEM((K, 2 * D + 6), _BF),
            pltpu.VMEM((2 * D, K), _BF),
            pltpu.VMEM((K, 1), _F32),
        ],
    )(z, W, emb)


# speedup vs baseline: 1.0162x; 1.0017x over previous
import jax
import jax.numpy as jnp
from jax.experimental import pallas as pl
from jax.experimental.pallas import tpu as pltpu

B, C_IN, N_T = 4, 384, 196
K, D = 1024, 64

_BF = jnp.bfloat16
_F32 = jnp.float32


def _split3(x):
    x0 = x.astype(_BF)
    r1 = x - x0.astype(_F32)
    x1 = r1.astype(_BF)
    x2 = (r1 - x1.astype(_F32)).astype(_BF)
    return x0, x1, x2


def _vq_body(z_ref, w_ref, emb_ref, out_ref, a1_ref, a2_ref, at_ref, en_ref):
    @pl.when(pl.program_id(0) == 0)
    def _prep():
        emb = emb_ref[...]
        emb2 = jnp.sum(emb * emb, axis=1, keepdims=True)
        en_ref[...] = jnp.sqrt(emb2)
        e0, e1, e2 = _split3(emb)
        m2e0 = (-2.0 * e0.astype(_F32)).astype(_BF)
        m2e1 = (-2.0 * e1.astype(_F32)).astype(_BF)
        m2e2 = (-2.0 * e2.astype(_F32)).astype(_BF)
        a1_ref[...] = jnp.concatenate([m2e0, m2e1, m2e2, m2e0], axis=1)
        q0, q1, q2 = _split3(emb2)
        onesk = jnp.ones((K, 3), _BF)
        a2_ref[...] = jnp.concatenate([m2e0, m2e1, q0.astype(_BF),
                                       q1.astype(_BF), q2.astype(_BF),
                                       onesk], axis=1)
        et0, et1, _ = _split3(emb.T)
        at_ref[...] = jnp.concatenate([et0, et1], axis=0)

    a1 = a1_ref[...]
    a2 = a2_ref[...]
    emb_norm = en_ref[...]

    iota_k16 = jax.lax.broadcasted_iota(jnp.int16, (K, N_T), 0)
    KC = 128
    iota_c = jax.lax.broadcasted_iota(jnp.int32, (KC, N_T), 0)

    zb = z_ref[0]
    ze = jnp.dot(w_ref[...], zb)
    ze2 = jnp.sum(ze * ze, axis=0, keepdims=True)
    s0 = ze.astype(_BF)
    r1 = ze - s0.astype(_F32)
    s1 = r1.astype(_BF)
    r2 = r1 - s1.astype(_F32)
    t0, t1, t2 = _split3(ze2)
    one_n = jnp.ones((1, N_T), _F32)
    x1 = jnp.concatenate([ze, ze, ze, r2], axis=0).astype(_BF)
    x2 = jnp.concatenate([r1, r1, one_n, one_n, one_n,
                          t0.astype(_F32), t1.astype(_F32),
                          t2.astype(_F32)], axis=0).astype(_BF)
    num2 = (jnp.dot(a1, x1, preferred_element_type=_F32)
            + jnp.dot(a2, x2, preferred_element_type=_F32))
    a_n = jnp.sqrt(ze2)
    mval = None
    for c in range(K // KC):
        den_c = a_n + emb_norm[c * KC:(c + 1) * KC]
        s2c = num2[c * KC:(c + 1) * KC] / (den_c * den_c)
        cmin = jnp.min(s2c, axis=0, keepdims=True)
        cidx = jnp.min(jnp.where(s2c == cmin, iota_c, KC),
                       axis=0, keepdims=True) + (c * KC)
        if mval is None:
            mval, midx = cmin, cidx
        else:
            upd = cmin < mval
            midx = jnp.where(upd, cidx, midx)
            mval = jnp.minimum(mval, cmin)
    onehot = (iota_k16 == midx.astype(jnp.int16)).astype(_BF)
    zq2 = jnp.dot(at_ref[...], onehot,
                  preferred_element_type=_F32)
    zq = zq2[:D] + zq2[D:]
    out_ref[0] = ze + (zq - ze)


@jax.jit
def kernel(z, W, emb):
    return pl.pallas_call(
        _vq_body,
        grid=(B,),
        in_specs=[
            pl.BlockSpec((1, C_IN, N_T), lambda b: (b, 0, 0)),
            pl.BlockSpec((D, C_IN), lambda b: (0, 0)),
            pl.BlockSpec((K, D), lambda b: (0, 0)),
        ],
        out_specs=pl.BlockSpec((1, D, N_T), lambda b: (b, 0, 0)),
        out_shape=jax.ShapeDtypeStruct((B, D, N_T), jnp.float32),
        scratch_shapes=[
            pltpu.VMEM((K, 4 * D), _BF),
            pltpu.VMEM((K, 2 * D + 6), _BF),
            pltpu.VMEM((2 * D, K), _BF),
            pltpu.VMEM((K, 1), _F32),
        ],
    )(z, W, emb)


# speedup vs baseline: 1.0292x; 1.0128x over previous
import jax
import jax.numpy as jnp
from jax.experimental import pallas as pl
from jax.experimental.pallas import tpu as pltpu

B, C_IN, N_T = 4, 384, 196
K, D = 1024, 64

_BF = jnp.bfloat16
_F32 = jnp.float32


def _split3(x):
    x0 = x.astype(_BF)
    r1 = x - x0.astype(_F32)
    x1 = r1.astype(_BF)
    x2 = (r1 - x1.astype(_F32)).astype(_BF)
    return x0, x1, x2


def _vq_body(z_ref, w_ref, emb_ref, out_ref, a1_ref, a2_ref, at_ref, en_ref):
    @pl.when(pl.program_id(0) == 0)
    def _prep():
        emb = emb_ref[...]
        emb2 = jnp.sum(emb * emb, axis=1, keepdims=True)
        en_ref[...] = jnp.sqrt(emb2)
        e0, e1, e2 = _split3(emb)
        m2e0 = (-2.0 * e0.astype(_F32)).astype(_BF)
        m2e1 = (-2.0 * e1.astype(_F32)).astype(_BF)
        m2e2 = (-2.0 * e2.astype(_F32)).astype(_BF)
        a1_ref[...] = jnp.concatenate([m2e0, m2e1, m2e2, m2e0], axis=1)
        q0, q1, q2 = _split3(emb2)
        onesk = jnp.ones((K, 3), _BF)
        a2_ref[...] = jnp.concatenate([m2e0, m2e1, q0.astype(_BF),
                                       q1.astype(_BF), q2.astype(_BF),
                                       onesk], axis=1)
        et0, et1, _ = _split3(emb.T)
        at_ref[...] = jnp.concatenate([et0, et1], axis=0)

    a1 = a1_ref[...]
    a2 = a2_ref[...]
    emb_norm = en_ref[...]

    iota_k = jax.lax.broadcasted_iota(jnp.int32, (K, N_T), 0)
    KC = 128
    iota_c = jax.lax.broadcasted_iota(jnp.int32, (KC, N_T), 0)

    zb = z_ref[0]
    ze = jnp.dot(w_ref[...], zb)
    ze2 = jnp.sum(ze * ze, axis=0, keepdims=True)
    s0 = ze.astype(_BF)
    r1 = ze - s0.astype(_F32)
    s1 = r1.astype(_BF)
    r2 = r1 - s1.astype(_F32)
    t0, t1, t2 = _split3(ze2)
    one_n = jnp.ones((1, N_T), _F32)
    x1 = jnp.concatenate([ze, ze, ze, r2], axis=0).astype(_BF)
    x2 = jnp.concatenate([r1, r1, one_n, one_n, one_n,
                          t0.astype(_F32), t1.astype(_F32),
                          t2.astype(_F32)], axis=0).astype(_BF)
    num2 = (jnp.dot(a1, x1, preferred_element_type=_F32)
            + jnp.dot(a2, x2, preferred_element_type=_F32))
    a_n = jnp.sqrt(ze2)
    mval = None
    for c in range(K // KC):
        den_c = a_n + emb_norm[c * KC:(c + 1) * KC]
        s2c = num2[c * KC:(c + 1) * KC] / (den_c * den_c)
        cmin = jnp.min(s2c, axis=0, keepdims=True)
        cidx = jnp.min(jnp.where(s2c == cmin, iota_c, KC),
                       axis=0, keepdims=True) + (c * KC)
        if mval is None:
            mval, midx = cmin, cidx
        else:
            upd = cmin < mval
            midx = jnp.where(upd, cidx, midx)
            mval = jnp.minimum(mval, cmin)
    onehot = (iota_k == midx).astype(_BF)
    zq2 = jnp.dot(at_ref[...], onehot,
                  preferred_element_type=_F32)
    zq = zq2[:D] + zq2[D:]
    out_ref[0] = ze + (zq - ze)


@jax.jit
def kernel(z, W, emb):
    return pl.pallas_call(
        _vq_body,
        grid=(B,),
        in_specs=[
            pl.BlockSpec((1, C_IN, N_T), lambda b: (b, 0, 0)),
            pl.BlockSpec((D, C_IN), lambda b: (0, 0)),
            pl.BlockSpec((K, D), lambda b: (0, 0)),
        ],
        out_specs=pl.BlockSpec((1, D, N_T), lambda b: (b, 0, 0)),
        out_shape=jax.ShapeDtypeStruct((B, D, N_T), jnp.float32),
        scratch_shapes=[
            pltpu.VMEM((K, 4 * D), _BF),
            pltpu.VMEM((K, 2 * D + 6), _BF),
            pltpu.VMEM((2 * D, K), _BF),
            pltpu.VMEM((K, 1), _F32),
        ],
    )(z, W, emb)
